# baseline (speedup 1.0000x reference)
import functools

import jax
import jax.numpy as jnp
from jax import lax
from jax.experimental import pallas as pl
from jax.experimental.pallas import tpu as pltpu
from jax.experimental.pallas import tpu_sc as plsc


@functools.lru_cache(maxsize=None)
def _make_gather(V, D, B):
    info = plsc.get_sparse_core_info()
    NC, NS = info.num_cores, info.num_subcores
    NW = NC * NS
    assert B % (NW * 128) == 0
    b_per_w = B // NW
    K = 8
    CH = K * 128
    G = b_per_w // CH
    assert b_per_w % CH == 0
    rows_w = b_per_w // 128

    mesh = plsc.VectorSubcoreMesh(core_axis_name="c", subcore_axis_name="s")

    @functools.partial(
        pl.kernel,
        mesh=mesh,
        out_type=jax.ShapeDtypeStruct((B, D), jnp.float32),
        compiler_params=pltpu.CompilerParams(use_tc_tiling_on_sc=False),
        scratch_types=[
            pltpu.VMEM((K, 128), jnp.int32),
            pltpu.VMEM((CH, D), jnp.float32),
            pltpu.SemaphoreType.DMA,
        ],
    )
    def gather_kernel(table_hbm, idx_hbm, out_hbm, idx_v, rows_v, sem):
        wid = lax.axis_index("s") * NC + lax.axis_index("c")
        idx_row0 = wid * rows_w
        out_row0 = wid * b_per_w

        def body(g, carry):
            pltpu.sync_copy(idx_hbm.at[pl.ds(idx_row0 + g * K, K)], idx_v)
            cps = [
                pltpu.async_copy(
                    table_hbm.at[idx_v.at[j]],
                    rows_v.at[pl.ds(j * 128, 128)],
                    sem,
                )
                for j in range(K)
            ]
            for cp in cps:
                cp.wait()
            pltpu.sync_copy(rows_v, out_hbm.at[pl.ds(out_row0 + g * CH, CH)])
            return carry

        lax.fori_loop(0, G, body, 0)

    return gather_kernel


def kernel(input_ids, table):
    Bt, T = input_ids.shape
    B = Bt * T
    V, D = table.shape
    idx2d = input_ids.reshape(B // 128, 128).astype(jnp.int32)
    out = _make_gather(V, D, B)(table, idx2d)
    return out.reshape(Bt, T, D)


# speedup vs baseline: 1.0124x; 1.0124x over previous
import functools

import jax
import jax.numpy as jnp
from jax import lax
from jax.experimental import pallas as pl
from jax.experimental.pallas import tpu as pltpu
from jax.experimental.pallas import tpu_sc as plsc


@functools.lru_cache(maxsize=None)
def _make_gather(V, D, B):
    info = plsc.get_sparse_core_info()
    NC, NS = info.num_cores, info.num_subcores
    NW = NC * NS
    assert B % (NW * 128) == 0
    b_per_w = B // NW
    K = 4
    CH = K * 128
    G = b_per_w // CH
    assert b_per_w % CH == 0 and G % 2 == 0
    rows_w = b_per_w // 128

    mesh = plsc.VectorSubcoreMesh(core_axis_name="c", subcore_axis_name="s")

    @functools.partial(
        pl.kernel,
        mesh=mesh,
        out_type=jax.ShapeDtypeStruct((B, D), jnp.float32),
        compiler_params=pltpu.CompilerParams(use_tc_tiling_on_sc=False),
        scratch_types=[
            pltpu.VMEM((2, K, 128), jnp.int32),
            pltpu.VMEM((2, CH, D), jnp.float32),
            pltpu.SemaphoreType.DMA,
            pltpu.SemaphoreType.DMA,
            pltpu.SemaphoreType.DMA,
            pltpu.SemaphoreType.DMA,
            pltpu.SemaphoreType.DMA,
            pltpu.SemaphoreType.DMA,
        ],
    )
    def gather_kernel(table_hbm, idx_hbm, out_hbm, idx_v, rows_v,
                      si0, si1, sg0, sg1, so0, so1):
        si = (si0, si1)
        sg = (sg0, sg1)
        so = (so0, so1)
        wid = lax.axis_index("s") * NC + lax.axis_index("c")
        idx_row0 = wid * rows_w
        out_row0 = wid * b_per_w

        def idx_start(g, b):
            pltpu.async_copy(
                idx_hbm.at[pl.ds(idx_row0 + g * K, K)], idx_v.at[b], si[b])

        def idx_wait(b):
            pltpu.make_async_copy(
                idx_hbm.at[pl.ds(0, K)], idx_v.at[b], si[b]).wait()

        def out_drain(b):
            pltpu.make_async_copy(
                out_hbm.at[pl.ds(0, CH)], rows_v.at[b], so[b]).wait()

        idx_start(0, 0)
        idx_start(1, 1)

        def body(i, carry):
            for b in range(2):
                @pl.when(i > 0)
                def _():
                    out_drain(b)
                idx_wait(b)
                for j in range(K):
                    pltpu.async_copy(
                        table_hbm.at[idx_v.at[b, j]],
                        rows_v.at[b, pl.ds(j * 128, 128)],
                        sg[b])
            for b in range(2):
                g = 2 * i + b
                for j in range(K):
                    pltpu.make_async_copy(
                        table_hbm.at[idx_v.at[b, j]],
                        rows_v.at[b, pl.ds(j * 128, 128)],
                        sg[b]).wait()
                pltpu.async_copy(
                    rows_v.at[b], out_hbm.at[pl.ds(out_row0 + g * CH, CH)],
                    so[b])

                @pl.when(g + 2 < G)
                def _():
                    idx_start(g + 2, b)
            return carry

        lax.fori_loop(0, G // 2, body, 0)
        out_drain(0)
        out_drain(1)

    return gather_kernel


def kernel(input_ids, table):
    Bt, T = input_ids.shape
    B = Bt * T
    V, D = table.shape
    idx2d = input_ids.reshape(B // 128, 128).astype(jnp.int32)
    out = _make_gather(V, D, B)(table, idx2d)
    return out.reshape(Bt, T, D)
